# Initial kernel scaffold; baseline (speedup 1.0000x reference)
import jax
import jax.numpy as jnp
from jax.experimental import pallas as pl


def kernel(center, outside, all_vocabs, W_center, W_outside):
    raise NotImplementedError("write your pallas kernel here")


# speedup vs baseline: 68.1315x; 68.1315x over previous
import functools

import jax
import jax.numpy as jnp
from jax import lax
from jax.experimental import pallas as pl
from jax.experimental.pallas import tpu as pltpu
from jax.experimental.pallas import tpu_sc as plsc

BATCH = 1024
VOCAB = 1000
EMB = 64
VPAD = 1024
LANES = 16
NC = 2
NS = 16
NW = NC * NS
BPW = BATCH // NW


def _sc_gather_body(cidx_hbm, oidx_hbm, wcomb_hbm,
                    ce_hbm, oe_hbm,
                    idx_v, rows_v, sem):
    wid = lax.axis_index("s") * NC + lax.axis_index("c")
    base = wid * BPW
    pltpu.sync_copy(cidx_hbm.at[pl.ds(base, BPW)], idx_v)
    pltpu.async_copy(wcomb_hbm.at[idx_v], rows_v, sem).wait()
    pltpu.sync_copy(rows_v, ce_hbm.at[pl.ds(base, BPW)])
    pltpu.sync_copy(oidx_hbm.at[pl.ds(base, BPW)], idx_v)
    pltpu.async_copy(wcomb_hbm.at[idx_v], rows_v, sem).wait()
    pltpu.sync_copy(rows_v, oe_hbm.at[pl.ds(base, BPW)])


_sc_gather = pl.kernel(
    _sc_gather_body,
    out_type=[jax.ShapeDtypeStruct((BATCH, 2 * EMB), jnp.float32),
              jax.ShapeDtypeStruct((BATCH, 2 * EMB), jnp.float32)],
    mesh=plsc.VectorSubcoreMesh(core_axis_name="c", subcore_axis_name="s"),
    scratch_types=[pltpu.VMEM((BPW,), jnp.int32),
                   pltpu.VMEM((BPW, 2 * EMB), jnp.float32),
                   pltpu.SemaphoreType.DMA],
)


def _tc_main_body(ce_ref, oe_ref, wo_ref, es_ref, top_ref):
    ce = ce_ref[:, :EMB]
    oe = oe_ref[:, EMB:]
    wo = wo_ref[...]
    s = lax.dot_general(ce, wo, (((1,), (1,)), ((), ())),
                        preferred_element_type=jnp.float32)
    col = lax.broadcasted_iota(jnp.int32, (BATCH, VPAD), 1)
    es_ref[...] = jnp.where(col < VOCAB, jnp.exp(s), 0.0)
    top_ref[...] = jnp.sum(ce * oe, axis=1, keepdims=True)


_tc_main = pl.pallas_call(
    _tc_main_body,
    out_shape=[jax.ShapeDtypeStruct((BATCH, VPAD), jnp.float32),
               jax.ShapeDtypeStruct((BATCH, 1), jnp.float32)],
)


def _sc_lowsum_body(av_hbm, es_hbm, lp_hbm, av_v, es_v, lp_v, sem):
    wid = lax.axis_index("s") * NC + lax.axis_index("c")
    base = wid * BPW * VPAD
    c1 = pltpu.async_copy(av_hbm.at[pl.ds(base, BPW * VPAD)], av_v, sem)
    c2 = pltpu.async_copy(es_hbm.at[pl.ds(base, BPW * VPAD)], es_v, sem)
    c1.wait()
    c2.wait()

    def row_body(r, _):
        roff = r * VPAD

        def chunk_body(j, acc):
            a = av_v[pl.ds(roff + j * LANES, LANES)]
            g = plsc.load_gather(es_v, [a + roff])
            return acc + g

        acc = lax.fori_loop(0, VPAD // LANES, chunk_body,
                            jnp.zeros((LANES,), jnp.float32))
        lp_v[pl.ds(r * LANES, LANES)] = acc
        return 0

    lax.fori_loop(0, BPW, row_body, 0)
    pltpu.sync_copy(lp_v, lp_hbm.at[pl.ds(wid * BPW * LANES, BPW * LANES)])


_sc_lowsum = pl.kernel(
    _sc_lowsum_body,
    out_type=jax.ShapeDtypeStruct((BATCH * LANES,), jnp.float32),
    mesh=plsc.VectorSubcoreMesh(core_axis_name="c", subcore_axis_name="s"),
    scratch_types=[pltpu.VMEM((BPW * VPAD,), jnp.int32),
                   pltpu.VMEM((BPW * VPAD,), jnp.float32),
                   pltpu.VMEM((BPW * LANES,), jnp.float32),
                   pltpu.SemaphoreType.DMA],
    compiler_params=pltpu.CompilerParams(use_tc_tiling_on_sc=False,
                                         needs_layout_passes=False),
)


def _tc_final_body(lp_ref, top_ref, out_ref):
    low = jnp.sum(lp_ref[...], axis=1, keepdims=True)
    val = jnp.log(low) - top_ref[...]
    out_ref[...] = (jnp.sum(val) / BATCH).reshape(1, 1)


_tc_final = pl.pallas_call(
    _tc_final_body,
    out_shape=jax.ShapeDtypeStruct((1, 1), jnp.float32),
)


def kernel(center, outside, all_vocabs, W_center, W_outside):
    c_idx = center.reshape(BATCH)
    o_idx = outside.reshape(BATCH)
    av_pad = jnp.pad(all_vocabs, ((0, 0), (0, VPAD - VOCAB)),
                     constant_values=VOCAB)
    wo_pad = jnp.pad(W_outside, ((0, VPAD - VOCAB), (0, 0)))
    w_comb = jnp.concatenate([W_center, W_outside], axis=1)
    ce, oe = _sc_gather(c_idx, o_idx, w_comb)
    es, top = _tc_main(ce, oe, wo_pad)
    lp = _sc_lowsum(av_pad.reshape(BATCH * VPAD),
                    es.reshape(BATCH * VPAD))
    loss = _tc_final(lp.reshape(BATCH, LANES), top)
    return loss[0, 0]
